# baseline (speedup 1.0000x reference)
import functools

import jax
import jax.numpy as jnp
from jax import lax
from jax.experimental import pallas as pl
from jax.experimental.pallas import tpu as pltpu
from jax.experimental.pallas import tpu_sc as plsc

_INFO = plsc.get_sparse_core_info()
_NC, _NS, _L = _INFO.num_cores, _INFO.num_subcores, _INFO.num_lanes
_NW = _NC * _NS


def _make_sc_partials(B, V, S):
    rpt = B // _NW
    mesh = plsc.VectorSubcoreMesh(core_axis_name="c", subcore_axis_name="s")

    @functools.partial(
        pl.kernel,
        out_type=(
            jax.ShapeDtypeStruct((B,), jnp.float32),
            jax.ShapeDtypeStruct((B,), jnp.float32),
        ),
        mesh=mesh,
        compiler_params=pltpu.CompilerParams(needs_layout_passes=False),
        scratch_types=[
            pltpu.VMEM((rpt,), jnp.int32),
            pltpu.VMEM((S,), jnp.int32),
            pltpu.VMEM((V,), jnp.float32),
            pltpu.VMEM((S,), jnp.float32),
            pltpu.VMEM((rpt,), jnp.float32),
            pltpu.VMEM((rpt,), jnp.float32),
        ],
    )
    def sc_partials(x_hbm, tgt_hbm, smp_hbm, outA_hbm, outP_hbm,
                    tgt_v, sidx_v, row_v, s_v, oA_v, oP_v):
        wid = lax.axis_index("s") * _NC + lax.axis_index("c")
        base = wid * rpt
        pltpu.sync_copy(tgt_hbm.at[pl.ds(base, rpt)], tgt_v)
        lane0 = lax.iota(jnp.int32, _L) == 0

        def row_step(r, carry):
            b = base + r
            pltpu.sync_copy(x_hbm.at[b], row_v)
            pltpu.sync_copy(smp_hbm.at[b], sidx_v)

            rvec = jnp.full((_L,), r, jnp.int32)
            tidx = plsc.load_gather(tgt_v, [rvec])
            tvec = plsc.load_gather(row_v, [tidx])

            def p1(j, mvec):
                idx = sidx_v[pl.ds(j * _L, _L)]
                v = plsc.load_gather(row_v, [idx])
                s_v[pl.ds(j * _L, _L)] = v
                return jnp.maximum(mvec, v)

            mvec = lax.fori_loop(0, S // _L, p1,
                                 jnp.full((_L,), -jnp.inf, jnp.float32))
            m = lax.reduce_max(mvec, (0,))

            zero = jnp.zeros((_L,), jnp.float32)

            def p2(j, acc):
                accE, accA, accP = acc
                v = s_v[pl.ds(j * _L, _L)]
                e = jnp.exp(v - m)
                sig = 1.0 / (1.0 + jnp.exp(v - tvec))
                return (accE + e, accA + e * sig, accP + e * v * v)

            accE, accA, accP = lax.fori_loop(0, S // _L, p2,
                                             (zero, zero, zero))
            E = lax.reduce_sum(accE, (0,))
            A = lax.reduce_sum(accA, (0,))
            P = lax.reduce_sum(accP, (0,))

            Evec = jnp.full((_L,), E)
            plsc.store_scatter(oA_v, [rvec], jnp.full((_L,), A) / Evec, mask=lane0)
            plsc.store_scatter(oP_v, [rvec], jnp.full((_L,), P) / Evec, mask=lane0)
            return carry

        lax.fori_loop(0, rpt, row_step, 0)
        pltpu.sync_copy(oA_v, outA_hbm.at[pl.ds(base, rpt)])
        pltpu.sync_copy(oP_v, outP_hbm.at[pl.ds(base, rpt)])

    return sc_partials


def _finish(a, p):
    B = a.shape[0]
    a2 = a.reshape(8, B // 8)
    p2 = p.reshape(8, B // 8)

    def body(a_ref, p_ref, o_ref):
        o_ref[0, 0] = jnp.mean(-jnp.log(a_ref[...]) + p_ref[...])

    out = pl.pallas_call(
        body,
        out_shape=jax.ShapeDtypeStruct((1, 1), jnp.float32),
        out_specs=pl.BlockSpec(memory_space=pltpu.SMEM),
    )(a2, p2)
    return out[0, 0]


def kernel(input, target, samples):
    B, V = input.shape
    S = samples.shape[1]
    tgt = target.astype(jnp.int32)
    smp = samples.astype(jnp.int32)
    outA, outP = _make_sc_partials(B, V, S)(input, tgt, smp)
    return _finish(outA, outP)
